# Initial kernel scaffold; baseline (speedup 1.0000x reference)
import functools
import math

import jax
import jax.numpy as jnp
from jax.experimental import pallas as pl
from jax.experimental.pallas import tpu as pltpu

_H = 12
_CAPF = 1.25
_PREC = jax.lax.Precision.HIGHEST
_INTERPRET = False


def _embed_body(p_ref, w_ref, b_ref, o_ref):
    o_ref[...] = (
        jnp.dot(p_ref[...], w_ref[...], precision=_PREC,
                preferred_element_type=jnp.float32) + b_ref[...]
    )


def _embed(patches, conv_w, conv_b):
    n, _ = patches.shape
    d = conv_w.shape[1]
    return pl.pallas_call(
        _embed_body,
        out_shape=jax.ShapeDtypeStruct((n, d), jnp.float32),
        interpret=_INTERPRET,
    )(patches, conv_w, conv_b.reshape(1, d))


def _attn_body(h_ref, g_ref, b_ref, qkvw_ref, qkvb_ref, pw_ref, pb_ref,
               o_ref, *, heads, dk):
    h = h_ref[0]
    m = jnp.mean(h, -1, keepdims=True)
    v = jnp.mean((h - m) ** 2, -1, keepdims=True)
    t = (h - m) / jnp.sqrt(v + 1e-6) * g_ref[...] + b_ref[...]
    qkv = jnp.dot(t, qkvw_ref[...], precision=_PREC,
                  preferred_element_type=jnp.float32) + qkvb_ref[...]
    scale = 1.0 / math.sqrt(dk)
    acc = None
    for hd in range(heads):
        q = qkv[:, (0 * heads + hd) * dk:(0 * heads + hd + 1) * dk]
        k = qkv[:, (1 * heads + hd) * dk:(1 * heads + hd + 1) * dk]
        vv = qkv[:, (2 * heads + hd) * dk:(2 * heads + hd + 1) * dk]
        logits = jax.lax.dot_general(
            q, k, (((1,), (1,)), ((), ())), precision=_PREC,
            preferred_element_type=jnp.float32) * scale
        logits = logits - jnp.max(logits, -1, keepdims=True)
        e = jnp.exp(logits)
        att = e / jnp.sum(e, -1, keepdims=True)
        oh = jnp.dot(att, vv, precision=_PREC,
                     preferred_element_type=jnp.float32)
        part = jnp.dot(oh, pw_ref[hd * dk:(hd + 1) * dk, :], precision=_PREC,
                       preferred_element_type=jnp.float32)
        acc = part if acc is None else acc + part
    o_ref[0] = h + acc + pb_ref[...]


def _attention(h, ln_g, ln_b, qkv_w, qkv_b, proj_w, proj_b):
    bb, s, d = h.shape
    dk = qkv_w.shape[1] // (3 * _H)
    return pl.pallas_call(
        functools.partial(_attn_body, heads=_H, dk=dk),
        grid=(bb,),
        in_specs=[
            pl.BlockSpec((1, s, d), lambda b: (b, 0, 0)),
            pl.BlockSpec((1, d), lambda b: (0, 0)),
            pl.BlockSpec((1, d), lambda b: (0, 0)),
            pl.BlockSpec(qkv_w.shape, lambda b: (0, 0)),
            pl.BlockSpec((1, qkv_b.shape[0]), lambda b: (0, 0)),
            pl.BlockSpec(proj_w.shape, lambda b: (0, 0)),
            pl.BlockSpec((1, d), lambda b: (0, 0)),
        ],
        out_specs=pl.BlockSpec((1, s, d), lambda b: (b, 0, 0)),
        out_shape=jax.ShapeDtypeStruct((bb, s, d), jnp.float32),
        interpret=_INTERPRET,
    )(h, ln_g.reshape(1, d), ln_b.reshape(1, d), qkv_w,
      qkv_b.reshape(1, -1), proj_w, proj_b.reshape(1, d))


def _cumsum0(x):
    n = x.shape[0]
    s = 1
    while s < n:
        shifted = jnp.concatenate(
            [jnp.zeros((s, x.shape[1]), x.dtype), x[:-s]], axis=0)
        x = x + shifted
        s *= 2
    return x


def _route_body(h_ref, g_ref, b_ref, gw_ref,
                tln_ref, i1_ref, p1_ref, g1_ref, i2_ref, p2_ref, g2_ref,
                *, n_exp, cap):
    h = h_ref[...]
    m = jnp.mean(h, -1, keepdims=True)
    v = jnp.mean((h - m) ** 2, -1, keepdims=True)
    t = (h - m) / jnp.sqrt(v + 1e-6) * g_ref[...] + b_ref[...]
    tln_ref[...] = t
    logits = jnp.dot(t, gw_ref[...], precision=_PREC,
                     preferred_element_type=jnp.float32)
    logits = logits - jnp.max(logits, -1, keepdims=True)
    ee = jnp.exp(logits)
    probs = ee / jnp.sum(ee, -1, keepdims=True)
    lane = jax.lax.broadcasted_iota(jnp.int32, probs.shape, 1)
    mx1 = jnp.max(probs, -1, keepdims=True)
    idx1 = jnp.min(jnp.where(probs == mx1, lane, n_exp), -1, keepdims=True)
    m1 = (lane == idx1).astype(jnp.float32)
    probs2 = probs * (1.0 - m1)
    mx2 = jnp.max(probs2, -1, keepdims=True)
    idx2 = jnp.min(jnp.where(probs2 == mx2, lane, n_exp), -1, keepdims=True)
    m2 = (lane == idx2).astype(jnp.float32)
    loc1 = _cumsum0(m1) - m1
    loc2 = _cumsum0(m2) - m2 + jnp.sum(m1, 0, keepdims=True)
    m1k = m1 * (loc1 < cap)
    m2k = m2 * (loc2 < cap)
    pos1 = jnp.sum(loc1 * m1k, -1, keepdims=True)
    pos2 = jnp.sum(loc2 * m2k, -1, keepdims=True)
    s1 = jnp.sum(m1k, -1, keepdims=True)
    s2 = jnp.sum(m2k, -1, keepdims=True)
    g1 = jnp.sum(probs * m1k, -1, keepdims=True)
    g2 = jnp.sum(probs * m2k, -1, keepdims=True)
    den = g1 + g2 + 1e-9
    g1 = g1 / den * s1
    g2 = g2 / den * s2
    i1_ref[...] = jnp.where(s1 > 0, idx1, n_exp).astype(jnp.int32)
    i2_ref[...] = jnp.where(s2 > 0, idx2, n_exp).astype(jnp.int32)
    p1_ref[...] = pos1.astype(jnp.int32)
    p2_ref[...] = pos2.astype(jnp.int32)
    g1_ref[...] = g1
    g2_ref[...] = g2


def _route(h2, ln_g, ln_b, gate_w, cap):
    n, d = h2.shape
    n_exp = gate_w.shape[1]
    outs = (
        jax.ShapeDtypeStruct((n, d), jnp.float32),
        jax.ShapeDtypeStruct((n, 1), jnp.int32),
        jax.ShapeDtypeStruct((n, 1), jnp.int32),
        jax.ShapeDtypeStruct((n, 1), jnp.float32),
        jax.ShapeDtypeStruct((n, 1), jnp.int32),
        jax.ShapeDtypeStruct((n, 1), jnp.int32),
        jax.ShapeDtypeStruct((n, 1), jnp.float32),
    )
    return pl.pallas_call(
        functools.partial(_route_body, n_exp=n_exp, cap=cap),
        out_shape=outs,
        interpret=_INTERPRET,
    )(h2, ln_g.reshape(1, d), ln_b.reshape(1, d), gate_w)


def _ffn_body(tln_ref, h_ref, i1_ref, p1_ref, g1_ref, i2_ref, p2_ref, g2_ref,
              w1_ref, b1_ref, w2_ref, b2_ref, o_ref, ein_ref, *, cap_pad):
    e = pl.program_id(0)
    f = pl.program_id(1)
    n = tln_ref.shape[0]
    slot = jax.lax.broadcasted_iota(jnp.int32, (n, cap_pad), 1)
    oh1 = jnp.where((i1_ref[...] == e) & (slot == p1_ref[...]), 1.0, 0.0)
    oh2 = jnp.where((i2_ref[...] == e) & (slot == p2_ref[...]), 1.0, 0.0)
    comb = g1_ref[...] * oh1 + g2_ref[...] * oh2

    @pl.when(f == 0)
    def _():
        disp = oh1 + oh2
        ein_ref[...] = jax.lax.dot_general(
            disp, tln_ref[...], (((0,), (0,)), ((), ())), precision=_PREC,
            preferred_element_type=jnp.float32)

    ein = ein_ref[...]
    hh = jnp.dot(ein, w1_ref[0], precision=_PREC,
                 preferred_element_type=jnp.float32) + b1_ref[...]
    hh = jax.nn.gelu(hh, approximate=False)
    eo = jnp.dot(hh, w2_ref[0], precision=_PREC,
                 preferred_element_type=jnp.float32)
    part = jnp.dot(comb, eo, precision=_PREC,
                   preferred_element_type=jnp.float32)
    rowsum = jnp.sum(comb, -1, keepdims=True)
    bias_term = jnp.where(f == 0, 1.0, 0.0) * rowsum * b2_ref[...]
    part = part + bias_term
    first = (e == 0) & (f == 0)

    @pl.when(first)
    def _():
        o_ref[...] = h_ref[...] + part

    @pl.when(jnp.logical_not(first))
    def _():
        o_ref[...] = o_ref[...] + part


def _moe_ffn(tln, h2, i1, p1, g1, i2, p2, g2, w1, b1, w2, b2, cap_pad):
    n, d = tln.shape
    n_exp, _, dff = w1.shape
    fsplit = 2
    fch = dff // fsplit
    return pl.pallas_call(
        functools.partial(_ffn_body, cap_pad=cap_pad),
        grid=(n_exp, fsplit),
        in_specs=[
            pl.BlockSpec((n, d), lambda e, f: (0, 0)),
            pl.BlockSpec((n, d), lambda e, f: (0, 0)),
            pl.BlockSpec((n, 1), lambda e, f: (0, 0)),
            pl.BlockSpec((n, 1), lambda e, f: (0, 0)),
            pl.BlockSpec((n, 1), lambda e, f: (0, 0)),
            pl.BlockSpec((n, 1), lambda e, f: (0, 0)),
            pl.BlockSpec((n, 1), lambda e, f: (0, 0)),
            pl.BlockSpec((n, 1), lambda e, f: (0, 0)),
            pl.BlockSpec((1, d, fch), lambda e, f: (e, 0, f)),
            pl.BlockSpec((1, fch), lambda e, f: (e, f)),
            pl.BlockSpec((1, fch, d), lambda e, f: (e, f, 0)),
            pl.BlockSpec((1, d), lambda e, f: (e, 0)),
        ],
        out_specs=pl.BlockSpec((n, d), lambda e, f: (0, 0)),
        out_shape=jax.ShapeDtypeStruct((n, d), jnp.float32),
        scratch_shapes=[pltpu.VMEM((cap_pad, d), jnp.float32)],
        interpret=_INTERPRET,
    )(tln, h2, i1, p1, g1, i2, p2, g2, w1,
      b1.reshape(n_exp, dff), w2, b2.reshape(n_exp, d))


def _head_body(h_ref, g_ref, b_ref, cw_ref, cb_ref, o_ref, *, bb, s):
    h = h_ref[...]
    m = jnp.mean(h, -1, keepdims=True)
    v = jnp.mean((h - m) ** 2, -1, keepdims=True)
    t = (h - m) / jnp.sqrt(v + 1e-6) * g_ref[...] + b_ref[...]
    bi = jax.lax.broadcasted_iota(jnp.int32, (bb, bb * s), 0)
    ti = jax.lax.broadcasted_iota(jnp.int32, (bb, bb * s), 1)
    sel = (ti >= bi * s) & (ti < (bi + 1) * s)
    mm = jnp.where(sel, 1.0 / s, 0.0)
    pooled = jnp.dot(mm, t, precision=_PREC,
                     preferred_element_type=jnp.float32)
    o_ref[...] = jnp.dot(pooled, cw_ref[...], precision=_PREC,
                         preferred_element_type=jnp.float32) + cb_ref[...]


def _head(h2, bb, s, lnf_g, lnf_b, cls_w, cls_b):
    d = h2.shape[1]
    ncls = cls_w.shape[1]
    return pl.pallas_call(
        functools.partial(_head_body, bb=bb, s=s),
        out_shape=jax.ShapeDtypeStruct((bb, ncls), jnp.float32),
        interpret=_INTERPRET,
    )(h2, lnf_g.reshape(1, d), lnf_b.reshape(1, d), cls_w,
      cls_b.reshape(1, ncls))


def kernel(x, conv_w, conv_b, cls_token, pos_embed, qkv_w, qkv_b, proj_w,
           proj_b, ln1_g, ln1_b, ln2_g, ln2_b, gate_w, w1, b1, w2, b2,
           lnf_g, lnf_b, cls_w, cls_b):
    bb = x.shape[0]
    d = conv_w.shape[1]
    depth = ln1_g.shape[0]
    n_exp = w1.shape[0]
    p = int(round(math.sqrt(conv_w.shape[0] // 3)))
    g = x.shape[2] // p
    s = g * g + 1
    n = bb * s
    cap = int(math.ceil(_CAPF * 2 * n / n_exp))
    cap_pad = ((cap + 127) // 128) * 128

    patches = x.reshape(bb, 3, g, p, g, p).transpose(0, 2, 4, 1, 3, 5)
    patches = patches.reshape(bb * g * g, 3 * p * p)
    emb = _embed(patches, conv_w, conv_b).reshape(bb, g * g, d)
    cls = jnp.broadcast_to(cls_token, (bb, 1, d))
    h = jnp.concatenate([cls, emb], axis=1) + pos_embed

    for i in range(depth):
        h = _attention(h, ln1_g[i], ln1_b[i], qkv_w, qkv_b, proj_w, proj_b)
        h2 = h.reshape(n, d)
        tln, i1, p1, g1, i2, p2, g2 = _route(
            h2, ln2_g[i], ln2_b[i], gate_w[i], cap)
        h2 = _moe_ffn(tln, h2, i1, p1, g1, i2, p2, g2,
                      w1, b1, w2, b2, cap_pad)
        h = h2.reshape(bb, s, d)

    return _head(h.reshape(n, d), bb, s, lnf_g, lnf_b, cls_w, cls_b)


# speedup vs baseline: 1.4608x; 1.4608x over previous
import functools
import math

import jax
import jax.numpy as jnp
from jax.experimental import pallas as pl
from jax.experimental.pallas import tpu as pltpu

_H = 12
_CAPF = 1.25
_PREC = jax.lax.Precision.DEFAULT
_PREC_HI = jax.lax.Precision.HIGHEST
_INTERPRET = False


def _xsum(x):
    n, w = x.shape
    ckn = (w + 127) // 128
    wp = ckn * 128
    if wp != w:
        x = jnp.pad(x, ((0, 0), (0, wp - w)))
    acc = x[:, 0:128]
    for i in range(1, ckn):
        acc = acc + x[:, i * 128:(i + 1) * 128]
    rows = (min(w, 128) + 7) // 8 if ckn == 1 else 16
    a8 = acc[:, 0:8]
    for i in range(1, rows):
        a8 = a8 + acc[:, i * 8:(i + 1) * 8]
    a4 = a8[:, 0:4] + a8[:, 4:8]
    a2 = a4[:, 0:2] + a4[:, 2:4]
    return a2[:, 0:1] + a2[:, 1:2]


def _xsum_sm(x):
    n, w = x.shape
    ckn = (w + 127) // 128
    wp = ckn * 128
    if wp != w:
        x = jnp.pad(x, ((0, 0), (0, wp - w)))
    acc = x[:, 0:128]
    for i in range(1, ckn):
        acc = acc + x[:, i * 128:(i + 1) * 128]

    def shifted(y, k):
        return jnp.pad(y[:, k:], ((0, 0), (0, k)))

    a = acc
    for k in range(1, 8):
        a = a + shifted(acc, k)
    b1 = a
    for k in range(1, 8):
        b1 = b1 + shifted(a, 8 * k)
    return b1[:, 0:1] + b1[:, 64:65]


def _ln_in(h, g, b):
    w = h.shape[-1]
    m = _xsum(h) / w
    v = _xsum((h - m) ** 2) / w
    return (h - m) / jnp.sqrt(v + 1e-6) * g + b


def _embed_body(p_ref, w_ref, b_ref, o_ref):
    o_ref[...] = (
        jnp.dot(p_ref[...], w_ref[...], precision=_PREC,
                preferred_element_type=jnp.float32) + b_ref[...]
    )


def _embed(patches, conv_w, conv_b):
    n, _ = patches.shape
    d = conv_w.shape[1]
    return pl.pallas_call(
        _embed_body,
        out_shape=jax.ShapeDtypeStruct((n, d), jnp.float32),
        interpret=_INTERPRET,
    )(patches, conv_w, conv_b.reshape(1, d))


def _qkv_body(h_ref, g_ref, b_ref, qkvw_ref, qkvb_ref, o_ref):
    t = _ln_in(h_ref[...], g_ref[...], b_ref[...])
    o_ref[...] = jnp.dot(t, qkvw_ref[...], precision=_PREC,
                         preferred_element_type=jnp.float32) + qkvb_ref[...]


def _qkv(h2, ln_g, ln_b, qkv_w, qkv_b):
    n, d = h2.shape
    w3 = qkv_w.shape[1]
    return pl.pallas_call(
        _qkv_body,
        out_shape=jax.ShapeDtypeStruct((n, w3), jnp.float32),
        interpret=_INTERPRET,
    )(h2, ln_g.reshape(1, d), ln_b.reshape(1, d), qkv_w,
      qkv_b.reshape(1, w3))


def _attn_core_body(qkv_ref, o_ref, *, heads, dk):
    qkv = qkv_ref[0]
    scale = 1.0 / math.sqrt(dk)
    ohs = []
    for hd in range(heads):
        q = qkv[:, (0 * heads + hd) * dk:(0 * heads + hd + 1) * dk]
        k = qkv[:, (1 * heads + hd) * dk:(1 * heads + hd + 1) * dk]
        vv = qkv[:, (2 * heads + hd) * dk:(2 * heads + hd + 1) * dk]
        logits = jax.lax.dot_general(
            q, k, (((1,), (1,)), ((), ())), precision=_PREC,
            preferred_element_type=jnp.float32) * scale
        logits = logits - jnp.max(logits, -1, keepdims=True)
        e = jnp.exp(logits)
        att = e / _xsum_sm(e)
        ohs.append(jnp.dot(att, vv, precision=_PREC,
                           preferred_element_type=jnp.float32))
    o_ref[0] = jnp.concatenate(ohs, axis=1)


def _attn_core(qkv3, heads, dk):
    bb, s, w3 = qkv3.shape
    return pl.pallas_call(
        functools.partial(_attn_core_body, heads=heads, dk=dk),
        grid=(bb,),
        in_specs=[pl.BlockSpec((1, s, w3), lambda b: (b, 0, 0))],
        out_specs=pl.BlockSpec((1, s, heads * dk), lambda b: (b, 0, 0)),
        out_shape=jax.ShapeDtypeStruct((bb, s, heads * dk), jnp.float32),
        interpret=_INTERPRET,
    )(qkv3)


def _proj_body(o_ref, h_ref, pw_ref, pb_ref, out_ref):
    proj = jnp.dot(o_ref[...], pw_ref[...], precision=_PREC,
                   preferred_element_type=jnp.float32)
    out_ref[...] = h_ref[...] + proj + pb_ref[...]


def _proj_res(o2, h2, proj_w, proj_b):
    n, d = h2.shape
    return pl.pallas_call(
        _proj_body,
        out_shape=jax.ShapeDtypeStruct((n, d), jnp.float32),
        interpret=_INTERPRET,
    )(o2, h2, proj_w, proj_b.reshape(1, d))


def _attn_body(h_ref, g_ref, b_ref, qkvw_ref, qkvb_ref, pw_ref, pb_ref,
               o_ref, *, heads, dk):
    h = h_ref[0]
    t = _ln_in(h, g_ref[...], b_ref[...])
    qkv = jnp.dot(t, qkvw_ref[...], precision=_PREC,
                  preferred_element_type=jnp.float32) + qkvb_ref[...]
    scale = 1.0 / math.sqrt(dk)
    ohs = []
    for hd in range(heads):
        q = qkv[:, (0 * heads + hd) * dk:(0 * heads + hd + 1) * dk]
        k = qkv[:, (1 * heads + hd) * dk:(1 * heads + hd + 1) * dk]
        vv = qkv[:, (2 * heads + hd) * dk:(2 * heads + hd + 1) * dk]
        logits = jax.lax.dot_general(
            q, k, (((1,), (1,)), ((), ())), precision=_PREC,
            preferred_element_type=jnp.float32) * scale
        logits = logits - jnp.max(logits, -1, keepdims=True)
        e = jnp.exp(logits)
        att = e / _xsum_sm(e)
        ohs.append(jnp.dot(att, vv, precision=_PREC,
                           preferred_element_type=jnp.float32))
    o = jnp.concatenate(ohs, axis=1)
    proj = jnp.dot(o, pw_ref[...], precision=_PREC,
                   preferred_element_type=jnp.float32)
    o_ref[0] = h + proj + pb_ref[...]


def _attention(h, ln_g, ln_b, qkv_w, qkv_b, proj_w, proj_b):
    bb, s, d = h.shape
    dk = qkv_w.shape[1] // (3 * _H)
    return pl.pallas_call(
        functools.partial(_attn_body, heads=_H, dk=dk),
        grid=(bb,),
        in_specs=[
            pl.BlockSpec((1, s, d), lambda b: (b, 0, 0)),
            pl.BlockSpec((1, d), lambda b: (0, 0)),
            pl.BlockSpec((1, d), lambda b: (0, 0)),
            pl.BlockSpec(qkv_w.shape, lambda b: (0, 0)),
            pl.BlockSpec((1, qkv_b.shape[0]), lambda b: (0, 0)),
            pl.BlockSpec(proj_w.shape, lambda b: (0, 0)),
            pl.BlockSpec((1, d), lambda b: (0, 0)),
        ],
        out_specs=pl.BlockSpec((1, s, d), lambda b: (b, 0, 0)),
        out_shape=jax.ShapeDtypeStruct((bb, s, d), jnp.float32),
        interpret=_INTERPRET,
    )(h, ln_g.reshape(1, d), ln_b.reshape(1, d), qkv_w,
      qkv_b.reshape(1, -1), proj_w, proj_b.reshape(1, d))


def _cumsum0(x):
    n = x.shape[0]
    s = 1
    while s < n:
        shifted = jnp.concatenate(
            [jnp.zeros((s, x.shape[1]), x.dtype), x[:-s]], axis=0)
        x = x + shifted
        s *= 2
    return x


def _route_body(h_ref, g_ref, b_ref, gw_ref,
                tln_ref, i1_ref, p1_ref, g1_ref, i2_ref, p2_ref, g2_ref,
                *, n_exp, cap):
    h = h_ref[...]
    t = _ln_in(h, g_ref[...], b_ref[...])
    tln_ref[...] = t
    logits = jnp.dot(t, gw_ref[...], precision=_PREC,
                     preferred_element_type=jnp.float32)
    logits = logits - jnp.max(logits, -1, keepdims=True)
    ee = jnp.exp(logits)
    probs = ee / _xsum_sm(ee)
    lane = jax.lax.broadcasted_iota(jnp.int32, probs.shape, 1)
    mx1 = jnp.max(probs, -1, keepdims=True)
    idx1 = jnp.min(jnp.where(probs == mx1, lane, n_exp), -1, keepdims=True)
    m1 = (lane == idx1).astype(jnp.float32)
    probs2 = probs * (1.0 - m1)
    mx2 = jnp.max(probs2, -1, keepdims=True)
    idx2 = jnp.min(jnp.where(probs2 == mx2, lane, n_exp), -1, keepdims=True)
    m2 = (lane == idx2).astype(jnp.float32)
    loc1 = _cumsum0(m1) - m1
    loc2 = _cumsum0(m2) - m2 + jnp.sum(m1, 0, keepdims=True)
    m1k = m1 * (loc1 < cap)
    m2k = m2 * (loc2 < cap)
    pos1 = jnp.sum(loc1 * m1k, -1, keepdims=True)
    pos2 = jnp.sum(loc2 * m2k, -1, keepdims=True)
    s1 = jnp.sum(m1k, -1, keepdims=True)
    s2 = jnp.sum(m2k, -1, keepdims=True)
    g1 = jnp.sum(probs * m1k, -1, keepdims=True)
    g2 = jnp.sum(probs * m2k, -1, keepdims=True)
    den = g1 + g2 + 1e-9
    g1 = g1 / den * s1
    g2 = g2 / den * s2
    i1_ref[...] = jnp.where(s1 > 0, idx1, n_exp).astype(jnp.int32)
    i2_ref[...] = jnp.where(s2 > 0, idx2, n_exp).astype(jnp.int32)
    p1_ref[...] = pos1.astype(jnp.int32)
    p2_ref[...] = pos2.astype(jnp.int32)
    g1_ref[...] = g1
    g2_ref[...] = g2


def _route(h2, ln_g, ln_b, gate_w, cap):
    n, d = h2.shape
    n_exp = gate_w.shape[1]
    outs = (
        jax.ShapeDtypeStruct((n, d), jnp.float32),
        jax.ShapeDtypeStruct((n, 1), jnp.int32),
        jax.ShapeDtypeStruct((n, 1), jnp.int32),
        jax.ShapeDtypeStruct((n, 1), jnp.float32),
        jax.ShapeDtypeStruct((n, 1), jnp.int32),
        jax.ShapeDtypeStruct((n, 1), jnp.int32),
        jax.ShapeDtypeStruct((n, 1), jnp.float32),
    )
    return pl.pallas_call(
        functools.partial(_route_body, n_exp=n_exp, cap=cap),
        out_shape=outs,
        interpret=_INTERPRET,
    )(h2, ln_g.reshape(1, d), ln_b.reshape(1, d), gate_w)


def _disp_body(tln_ref, i1_ref, p1_ref, i2_ref, p2_ref, ein_ref, *, cap_pad):
    e = pl.program_id(0)
    n = tln_ref.shape[0]
    slot = jax.lax.broadcasted_iota(jnp.int32, (n, cap_pad), 1)
    oh1 = jnp.where((i1_ref[...] == e) & (slot == p1_ref[...]), 1.0, 0.0)
    oh2 = jnp.where((i2_ref[...] == e) & (slot == p2_ref[...]), 1.0, 0.0)
    ein_ref[0] = jax.lax.dot_general(
        oh1 + oh2, tln_ref[...], (((0,), (0,)), ((), ())), precision=_PREC,
        preferred_element_type=jnp.float32)


def _dispatch(tln, i1, p1, i2, p2, cap_pad, n_exp):
    n, d = tln.shape
    return pl.pallas_call(
        functools.partial(_disp_body, cap_pad=cap_pad),
        grid=(n_exp,),
        in_specs=[
            pl.BlockSpec((n, d), lambda e: (0, 0)),
            pl.BlockSpec((n, 1), lambda e: (0, 0)),
            pl.BlockSpec((n, 1), lambda e: (0, 0)),
            pl.BlockSpec((n, 1), lambda e: (0, 0)),
            pl.BlockSpec((n, 1), lambda e: (0, 0)),
        ],
        out_specs=pl.BlockSpec((1, cap_pad, d), lambda e: (e, 0, 0)),
        out_shape=jax.ShapeDtypeStruct((n_exp, cap_pad, d), jnp.float32),
        interpret=_INTERPRET,
    )(tln, i1, p1, i2, p2)


def _expert_body(ein_ref, w1_ref, b1_ref, w2_ref, b2_ref, eo_ref,
                 *, kc, fsplit):
    f = pl.program_id(1)
    hh = jnp.dot(ein_ref[0], w1_ref[0], precision=_PREC,
                 preferred_element_type=jnp.float32) + b1_ref[0]
    hh = hh * 0.5 * (1.0 + jax.lax.erf(hh * (1.0 / math.sqrt(2.0))))
    fch = hh.shape[1]
    for j in range(0, fch, kc):
        part = jnp.dot(hh[:, j:j + kc], w2_ref[0][j:j + kc, :],
                       precision=_PREC, preferred_element_type=jnp.float32)
        if j == 0:
            @pl.when(f == 0)
            def _():
                eo_ref[0] = part

            @pl.when(f != 0)
            def _():
                eo_ref[0] = eo_ref[0] + part
        else:
            eo_ref[0] = eo_ref[0] + part

    @pl.when(f == fsplit - 1)
    def _():
        eo_ref[0] = eo_ref[0] + b2_ref[0]


def _expert_ffn(ein, w1, b1, w2, b2, fsplit=2):
    n_exp, cap_pad, d = ein.shape
    dff = w1.shape[2]
    fch = dff // fsplit
    kc = min(768, fch)
    return pl.pallas_call(
        functools.partial(_expert_body, kc=kc, fsplit=fsplit),
        grid=(n_exp, fsplit),
        in_specs=[
            pl.BlockSpec((1, cap_pad, d), lambda e, f: (e, 0, 0)),
            pl.BlockSpec((1, d, fch), lambda e, f: (e, 0, f)),
            pl.BlockSpec((1, 1, fch), lambda e, f: (e, 0, f)),
            pl.BlockSpec((1, fch, d), lambda e, f: (e, f, 0)),
            pl.BlockSpec((1, 1, d), lambda e, f: (e, 0, 0)),
        ],
        out_specs=pl.BlockSpec((1, cap_pad, d), lambda e, f: (e, 0, 0)),
        out_shape=jax.ShapeDtypeStruct((n_exp, cap_pad, d), jnp.float32),
        interpret=_INTERPRET,
    )(ein, w1, b1.reshape(n_exp, 1, dff), w2, b2.reshape(n_exp, 1, d))


def _comb_body(h_ref, eo_ref, i1_ref, p1_ref, g1_ref, i2_ref, p2_ref, g2_ref,
               o_ref, *, cap_pad):
    e = pl.program_id(0)
    n = h_ref.shape[0]
    slot = jax.lax.broadcasted_iota(jnp.int32, (n, cap_pad), 1)
    oh1 = jnp.where((i1_ref[...] == e) & (slot == p1_ref[...]),
                    g1_ref[...], 0.0)
    oh2 = jnp.where((i2_ref[...] == e) & (slot == p2_ref[...]),
                    g2_ref[...], 0.0)
    part = jnp.dot(oh1 + oh2, eo_ref[0], precision=_PREC,
                   preferred_element_type=jnp.float32)

    @pl.when(e == 0)
    def _():
        o_ref[...] = h_ref[...] + part

    @pl.when(e != 0)
    def _():
        o_ref[...] = o_ref[...] + part


def _combine(h2, eo, i1, p1, g1, i2, p2, g2, cap_pad):
    n, d = h2.shape
    n_exp = eo.shape[0]
    return pl.pallas_call(
        functools.partial(_comb_body, cap_pad=cap_pad),
        grid=(n_exp,),
        in_specs=[
            pl.BlockSpec((n, d), lambda e: (0, 0)),
            pl.BlockSpec((1, cap_pad, d), lambda e: (e, 0, 0)),
            pl.BlockSpec((n, 1), lambda e: (0, 0)),
            pl.BlockSpec((n, 1), lambda e: (0, 0)),
            pl.BlockSpec((n, 1), lambda e: (0, 0)),
            pl.BlockSpec((n, 1), lambda e: (0, 0)),
            pl.BlockSpec((n, 1), lambda e: (0, 0)),
            pl.BlockSpec((n, 1), lambda e: (0, 0)),
        ],
        out_specs=pl.BlockSpec((n, d), lambda e: (0, 0)),
        out_shape=jax.ShapeDtypeStruct((n, d), jnp.float32),
        interpret=_INTERPRET,
    )(h2, eo, i1, p1, g1, i2, p2, g2)


def _moe_ffn(tln, h2, i1, p1, g1, i2, p2, g2, w1, b1, w2, b2, cap_pad):
    n, d = tln.shape
    ein = _dispatch(tln, i1, p1, i2, p2, cap_pad, w1.shape[0])
    eo = _expert_ffn(ein, w1, b1, w2, b2)
    return _combine(h2, eo, i1, p1, g1, i2, p2, g2, cap_pad)


def _head_body(h_ref, g_ref, b_ref, cw_ref, cb_ref, o_ref, *, bb, s):
    h = h_ref[...]
    t = _ln_in(h, g_ref[...], b_ref[...])
    bi = jax.lax.broadcasted_iota(jnp.int32, (bb, bb * s), 0)
    ti = jax.lax.broadcasted_iota(jnp.int32, (bb, bb * s), 1)
    sel = (ti >= bi * s) & (ti < (bi + 1) * s)
    mm = jnp.where(sel, 1.0 / s, 0.0)
    pooled = jnp.dot(mm, t, precision=_PREC_HI,
                     preferred_element_type=jnp.float32)
    o_ref[...] = jnp.dot(pooled, cw_ref[...], precision=_PREC,
                         preferred_element_type=jnp.float32) + cb_ref[...]


def _head(h2, bb, s, lnf_g, lnf_b, cls_w, cls_b):
    d = h2.shape[1]
    ncls = cls_w.shape[1]
    return pl.pallas_call(
        functools.partial(_head_body, bb=bb, s=s),
        out_shape=jax.ShapeDtypeStruct((bb, ncls), jnp.float32),
        interpret=_INTERPRET,
    )(h2, lnf_g.reshape(1, d), lnf_b.reshape(1, d), cls_w,
      cls_b.reshape(1, ncls))


def kernel(x, conv_w, conv_b, cls_token, pos_embed, qkv_w, qkv_b, proj_w,
           proj_b, ln1_g, ln1_b, ln2_g, ln2_b, gate_w, w1, b1, w2, b2,
           lnf_g, lnf_b, cls_w, cls_b):
    bb = x.shape[0]
    d = conv_w.shape[1]
    depth = ln1_g.shape[0]
    n_exp = w1.shape[0]
    p = int(round(math.sqrt(conv_w.shape[0] // 3)))
    g = x.shape[2] // p
    s = g * g + 1
    n = bb * s
    cap = int(math.ceil(_CAPF * 2 * n / n_exp))
    cap_pad = ((cap + 127) // 128) * 128

    patches = x.reshape(bb, 3, g, p, g, p).transpose(0, 2, 4, 1, 3, 5)
    patches = patches.reshape(bb * g * g, 3 * p * p)
    emb = _embed(patches, conv_w, conv_b).reshape(bb, g * g, d)
    cls = jnp.broadcast_to(cls_token, (bb, 1, d))
    h = jnp.concatenate([cls, emb], axis=1) + pos_embed

    for i in range(depth):
        h = _attention(h, ln1_g[i], ln1_b[i], qkv_w, qkv_b, proj_w, proj_b)
        h2 = h.reshape(n, d)
        tln, i1, p1, g1, i2, p2, g2 = _route(
            h2, ln2_g[i], ln2_b[i], gate_w[i], cap)
        h2 = _moe_ffn(tln, h2, i1, p1, g1, i2, p2, g2,
                      w1, b1, w2, b2, cap_pad)
        h = h2.reshape(bb, s, d)

    return _head(h.reshape(n, d), bb, s, lnf_g, lnf_b, cls_w, cls_b)
